# Initial kernel scaffold; baseline (speedup 1.0000x reference)
import jax
import jax.numpy as jnp
from jax.experimental import pallas as pl


def kernel(similarity_matrix):
    raise NotImplementedError("write your pallas kernel here")


# speedup vs baseline: 48.0629x; 48.0629x over previous
import jax
import jax.numpy as jnp
from jax.experimental import pallas as pl

_K = 16


def _bitonic_merge(planes):
    n = len(planes)
    if n == 1:
        return planes
    h = n // 2
    hi, lo = [], []
    for i in range(h):
        hi.append(jnp.maximum(planes[i], planes[i + h]))
        lo.append(jnp.minimum(planes[i], planes[i + h]))
    return _bitonic_merge(hi) + _bitonic_merge(lo)


def _sort_desc(planes):
    n = len(planes)
    if n == 1:
        return planes
    a = _sort_desc(planes[: n // 2])
    b = _sort_desc(planes[n // 2 :])
    return _bitonic_merge(a + b[::-1])


def _topk_body(x_ref, scores_ref, mask_ref):
    x = x_ref[0]
    P, C = x.shape
    R = P // _K

    planes = [x[i * R : (i + 1) * R, :] for i in range(_K)]
    planes = _sort_desc(planes)

    rows = R
    while rows > 1:
        h = rows // 2
        a = [p[:h] for p in planes]
        b = [p[h:] for p in planes]
        merged = [jnp.maximum(a[i], b[_K - 1 - i]) for i in range(_K)]
        planes = _bitonic_merge(merged)
        rows = h

    scores_ref[0, 0, :] = planes[0][0]
    t16 = planes[_K - 1]
    mask_ref[0] = jnp.where(x >= t16, jnp.float32(1.0), jnp.float32(0.0))


def kernel(similarity_matrix):
    B, P, C = similarity_matrix.shape
    scores3, mask = pl.pallas_call(
        _topk_body,
        grid=(B,),
        in_specs=[pl.BlockSpec((1, P, C), lambda b: (b, 0, 0))],
        out_specs=[
            pl.BlockSpec((1, 1, C), lambda b: (b, 0, 0)),
            pl.BlockSpec((1, P, C), lambda b: (b, 0, 0)),
        ],
        out_shape=[
            jax.ShapeDtypeStruct((B, 1, C), jnp.float32),
            jax.ShapeDtypeStruct((B, P, C), jnp.float32),
        ],
    )(similarity_matrix)
    return scores3.reshape(B, C), mask


# speedup vs baseline: 58.9578x; 1.2267x over previous
import jax
import jax.numpy as jnp
from jax.experimental import pallas as pl

_K = 16


def _batcher_pairs(n):
    pairs = []

    def merge(lo, m, r):
        step = r * 2
        if step < m:
            merge(lo, m, step)
            merge(lo + r, m, step)
            for i in range(lo + r, lo + m - r, step):
                pairs.append((i, i + r))
        else:
            pairs.append((lo, lo + r))

    def sort(lo, m):
        if m > 1:
            h = m // 2
            sort(lo, h)
            sort(lo + h, h)
            merge(lo, m, 1)

    sort(0, n)
    return pairs


_SORT16 = _batcher_pairs(_K)


def _sort_desc(v):
    v = list(v)
    for i, j in _SORT16:
        hi = jnp.maximum(v[i], v[j])
        lo = jnp.minimum(v[i], v[j])
        v[i], v[j] = hi, lo
    return v


def _bitonic_resort(v):
    n = len(v)
    if n == 1:
        return v
    h = n // 2
    hi = [jnp.maximum(v[i], v[i + h]) for i in range(h)]
    lo = [jnp.minimum(v[i], v[i + h]) for i in range(h)]
    return _bitonic_resort(hi) + _bitonic_resort(lo)


def _merge_top16(a, b):
    merged = [jnp.maximum(a[i], b[_K - 1 - i]) for i in range(_K)]
    return _bitonic_resort(merged)


def _topk_body(x_ref, scores_ref, mask_ref):
    P = x_ref.shape[1]
    C = x_ref.shape[2]
    CB = 128
    n_chunks = P // (8 * _K)

    for c in range(C // CB):
        csl = slice(c * CB, (c + 1) * CB)
        run = None
        for s in range(n_chunks):
            base = s * 8 * _K
            tiles = [
                x_ref[0, base + 8 * i : base + 8 * (i + 1), csl]
                for i in range(_K)
            ]
            tiles = _sort_desc(tiles)
            run = tiles if run is None else _merge_top16(run, tiles)
        for d in (1, 2, 4):
            rolled = [jnp.roll(run[_K - 1 - i], d, axis=0) for i in range(_K)]
            run = _bitonic_resort(
                [jnp.maximum(run[i], rolled[i]) for i in range(_K)]
            )
        scores_ref[0, 0:1, csl] = run[0][0:1]
        t16 = run[_K - 1]
        one = jnp.float32(1.0)
        zero = jnp.float32(0.0)
        for s in range(P // 8):
            xa = x_ref[0, 8 * s : 8 * (s + 1), csl]
            mask_ref[0, 8 * s : 8 * (s + 1), csl] = jnp.where(
                xa >= t16, one, zero
            )


def kernel(similarity_matrix):
    B, P, C = similarity_matrix.shape
    scores3, mask = pl.pallas_call(
        _topk_body,
        grid=(B,),
        in_specs=[pl.BlockSpec((1, P, C), lambda b: (b, 0, 0))],
        out_specs=[
            pl.BlockSpec((1, 1, C), lambda b: (b, 0, 0)),
            pl.BlockSpec((1, P, C), lambda b: (b, 0, 0)),
        ],
        out_shape=[
            jax.ShapeDtypeStruct((B, 1, C), jnp.float32),
            jax.ShapeDtypeStruct((B, P, C), jnp.float32),
        ],
    )(similarity_matrix)
    return scores3.reshape(B, C), mask


# speedup vs baseline: 68.4923x; 1.1617x over previous
import jax
import jax.numpy as jnp
from jax.experimental import pallas as pl

_K = 16
_BB = 2


def _batcher_pairs(n):
    pairs = []

    def merge(lo, m, r):
        step = r * 2
        if step < m:
            merge(lo, m, step)
            merge(lo + r, m, step)
            for i in range(lo + r, lo + m - r, step):
                pairs.append((i, i + r))
        else:
            pairs.append((lo, lo + r))

    def sort(lo, m):
        if m > 1:
            h = m // 2
            sort(lo, h)
            sort(lo + h, h)
            merge(lo, m, 1)

    sort(0, n)
    return pairs


_SORT16 = _batcher_pairs(_K)


def _sort_desc(v):
    v = list(v)
    for i, j in _SORT16:
        hi = jnp.maximum(v[i], v[j])
        lo = jnp.minimum(v[i], v[j])
        v[i], v[j] = hi, lo
    return v


def _bitonic_resort(v):
    n = len(v)
    if n == 1:
        return v
    h = n // 2
    hi = [jnp.maximum(v[i], v[i + h]) for i in range(h)]
    lo = [jnp.minimum(v[i], v[i + h]) for i in range(h)]
    return _bitonic_resort(hi) + _bitonic_resort(lo)


def _merge_top16(a, b):
    merged = [jnp.maximum(a[i], b[_K - 1 - i]) for i in range(_K)]
    return _bitonic_resort(merged)


def _topk_body(x_ref, scores_ref, mask_ref):
    P = x_ref.shape[1]
    C = x_ref.shape[2]
    CB = 128
    n_chunks = P // (8 * _K)

    for bi in range(_BB):
        for c in range(C // CB):
            csl = slice(c * CB, (c + 1) * CB)
            run = None
            for s in range(n_chunks):
                base = s * 8 * _K
                tiles = [
                    x_ref[bi, base + 8 * i : base + 8 * (i + 1), csl]
                    for i in range(_K)
                ]
                tiles = _sort_desc(tiles)
                run = tiles if run is None else _merge_top16(run, tiles)
            for d in (1, 2, 4):
                rolled = [
                    jnp.roll(run[_K - 1 - i], d, axis=0) for i in range(_K)
                ]
                run = _bitonic_resort(
                    [jnp.maximum(run[i], rolled[i]) for i in range(_K)]
                )
            scores_ref[bi, 0:1, csl] = run[0][0:1]
            t16 = run[_K - 1]
            one = jnp.float32(1.0)
            zero = jnp.float32(0.0)
            for s in range(P // 8):
                xa = x_ref[bi, 8 * s : 8 * (s + 1), csl]
                mask_ref[bi, 8 * s : 8 * (s + 1), csl] = jnp.where(
                    xa >= t16, one, zero
                )


def kernel(similarity_matrix):
    B, P, C = similarity_matrix.shape
    scores3, mask = pl.pallas_call(
        _topk_body,
        grid=(B // _BB,),
        in_specs=[pl.BlockSpec((_BB, P, C), lambda b: (b, 0, 0))],
        out_specs=[
            pl.BlockSpec((_BB, 1, C), lambda b: (b, 0, 0)),
            pl.BlockSpec((_BB, P, C), lambda b: (b, 0, 0)),
        ],
        out_shape=[
            jax.ShapeDtypeStruct((B, 1, C), jnp.float32),
            jax.ShapeDtypeStruct((B, P, C), jnp.float32),
        ],
    )(similarity_matrix)
    return scores3.reshape(B, C), mask


# speedup vs baseline: 73.1141x; 1.0675x over previous
import jax
import jax.numpy as jnp
from jax.experimental import pallas as pl

_K = 16
_BB = 4


def _batcher_pairs(n):
    pairs = []

    def merge(lo, m, r):
        step = r * 2
        if step < m:
            merge(lo, m, step)
            merge(lo + r, m, step)
            for i in range(lo + r, lo + m - r, step):
                pairs.append((i, i + r))
        else:
            pairs.append((lo, lo + r))

    def sort(lo, m):
        if m > 1:
            h = m // 2
            sort(lo, h)
            sort(lo + h, h)
            merge(lo, m, 1)

    sort(0, n)
    return pairs


_SORT16 = _batcher_pairs(_K)


def _sort_desc(v):
    v = list(v)
    for i, j in _SORT16:
        hi = jnp.maximum(v[i], v[j])
        lo = jnp.minimum(v[i], v[j])
        v[i], v[j] = hi, lo
    return v


def _bitonic_resort(v):
    n = len(v)
    if n == 1:
        return v
    h = n // 2
    hi = [jnp.maximum(v[i], v[i + h]) for i in range(h)]
    lo = [jnp.minimum(v[i], v[i + h]) for i in range(h)]
    return _bitonic_resort(hi) + _bitonic_resort(lo)


def _merge_top16(a, b):
    merged = [jnp.maximum(a[i], b[_K - 1 - i]) for i in range(_K)]
    return _bitonic_resort(merged)


def _topk_body(x_ref, scores_ref, mask_ref):
    P = x_ref.shape[1]
    C = x_ref.shape[2]
    CB = 128
    n_chunks = P // (8 * _K)

    for bi in range(_BB):
        for c in range(C // CB):
            csl = slice(c * CB, (c + 1) * CB)
            run = None
            for s in range(n_chunks):
                base = s * 8 * _K
                tiles = [
                    x_ref[bi, base + 8 * i : base + 8 * (i + 1), csl]
                    for i in range(_K)
                ]
                tiles = _sort_desc(tiles)
                run = tiles if run is None else _merge_top16(run, tiles)
            for d in (1, 2, 4):
                rolled = [
                    jnp.roll(run[_K - 1 - i], d, axis=0) for i in range(_K)
                ]
                run = _bitonic_resort(
                    [jnp.maximum(run[i], rolled[i]) for i in range(_K)]
                )
            scores_ref[bi, 0:1, csl] = run[0][0:1]
            t16 = run[_K - 1]
            one = jnp.float32(1.0)
            zero = jnp.float32(0.0)
            for s in range(P // 8):
                xa = x_ref[bi, 8 * s : 8 * (s + 1), csl]
                mask_ref[bi, 8 * s : 8 * (s + 1), csl] = jnp.where(
                    xa >= t16, one, zero
                )


def kernel(similarity_matrix):
    B, P, C = similarity_matrix.shape
    scores3, mask = pl.pallas_call(
        _topk_body,
        grid=(B // _BB,),
        in_specs=[pl.BlockSpec((_BB, P, C), lambda b: (b, 0, 0))],
        out_specs=[
            pl.BlockSpec((_BB, 1, C), lambda b: (b, 0, 0)),
            pl.BlockSpec((_BB, P, C), lambda b: (b, 0, 0)),
        ],
        out_shape=[
            jax.ShapeDtypeStruct((B, 1, C), jnp.float32),
            jax.ShapeDtypeStruct((B, P, C), jnp.float32),
        ],
    )(similarity_matrix)
    return scores3.reshape(B, C), mask
